# baseline (speedup 1.0000x reference)
import functools

import jax
import jax.numpy as jnp
from jax import lax
from jax.experimental import pallas as pl
from jax.experimental.pallas import tpu as pltpu
from jax.experimental.pallas import tpu_sc as plsc

NUM_NODES = 1000000
MEMORY_DIM = 64
BATCH = 16384

_NC = 2
_NS = 16
_NW = _NC * _NS
_BPW = BATCH // _NW
_L = 16
_CHUNK = 512
_SHIFT = 9
_OUTW = 128
_NUNITS = -(-NUM_NODES // _CHUNK)
_MAXCH = -(-_NUNITS // _NW) + 1
_NGRP = BATCH // _L
_PHYS_END = 1000064
_RING = 4

_mesh = plsc.VectorSubcoreMesh(core_axis_name="c", subcore_axis_name="s")


@functools.partial(
    pl.kernel,
    mesh=_mesh,
    out_type=(
        jax.ShapeDtypeStruct((BATCH, _OUTW), jnp.float32),
        jax.ShapeDtypeStruct((BATCH,), jnp.int32),
    ),
    scratch_types=[
        pltpu.VMEM((BATCH,), jnp.int32),
        pltpu.VMEM((BATCH,), jnp.int32),
        pltpu.VMEM((BATCH,), jnp.int32),
        pltpu.VMEM((2, MEMORY_DIM, _CHUNK), jnp.float32),
        pltpu.VMEM((_RING, _L, _OUTW), jnp.float32),
        pltpu.VMEM((64,), jnp.int32),
        pltpu.VMEM((64,), jnp.int32),
        pltpu.SMEM((64,), jnp.int32),
        pltpu.VMEM((_BPW,), jnp.int32),
        pltpu.VMEM((_BPW,), jnp.int32),
        pltpu.SemaphoreType.DMA,
        pltpu.SemaphoreType.DMA,
        pltpu.SemaphoreType.DMA,
    ],
    compiler_params=pltpu.CompilerParams(needs_layout_passes=False),
)
def _gather_sc(nid_hbm, memt_hbm, lu_hbm, out_hbm, lu_out,
               nid_v, srt_n, srt_p, blk_v, ring_v,
               cnts_v, fills_v, starts_sm, idx_lu, lu_v,
               sem_blk, sem_out, sem_lu):
    wid = lax.axis_index("s") * _NC + lax.axis_index("c")
    base = wid * _BPW
    u_lo = wid * _NUNITS // _NW
    u_hi = (wid + 1) * _NUNITS // _NW
    n_chunks = u_hi - u_lo
    lo_n = u_lo * _CHUNK
    hi_n = u_hi * _CHUNK

    lane_iota = lax.iota(jnp.int32, _L)
    zeros16 = jnp.zeros((_L,), jnp.int32)

    pltpu.sync_copy(nid_hbm.at[pl.ds(base, _BPW)], idx_lu)
    cp_lu = pltpu.async_copy(lu_hbm.at[idx_lu], lu_v, sem_lu)

    pltpu.sync_copy(nid_hbm, nid_v)

    def chunk_base(t):
        c0 = (u_lo + t) * _CHUNK
        return pl.multiple_of(jnp.minimum(c0, _PHYS_END - _CHUNK), 128)

    def issue(t, buf):
        pltpu.async_copy(
            memt_hbm.at[:, pl.ds(chunk_base(t), _CHUNK)], blk_v.at[buf],
            sem_blk)

    issue(0, 0)

    @pl.when(n_chunks > 1)
    def _():
        issue(1, 1)

    for i in range(4):
        cnts_v[pl.ds(i * _L, _L)] = zeros16

    def hist(g2, carry):
        for u in range(2):
            vals = nid_v[pl.ds((g2 * 2 + u) * _L, _L)]
            m = (vals >= lo_n) & (vals < hi_n)
            q = jnp.where(m, (vals - lo_n) >> _SHIFT, 0)
            cnt, lastm = plsc.scan_count(q, m)
            plsc.addupdate_scatter(cnts_v, [q], cnt, mask=lastm & m)
        return carry

    lax.fori_loop(0, _NGRP // 2, hist, 0)

    carry_in = jnp.int32(0)
    totals = []
    for i in range(4):
        ci = cnts_v[pl.ds(i * _L, _L)]
        si = plsc.cumsum(ci)
        ei = si - ci + jnp.broadcast_to(carry_in, (_L,))
        fills_v[pl.ds(i * _L, _L)] = ei
        carry_in = carry_in + jnp.squeeze(lax.slice(si, (_L - 1,), (_L,)))
        totals.append(ei)
    for i in range(4):
        for l in range(_L):
            starts_sm[i * _L + l] = jnp.squeeze(
                lax.slice(totals[i], (l,), (l + 1,)))

    def place(g2, carry):
        for u in range(2):
            g = g2 * 2 + u
            vals = nid_v[pl.ds(g * _L, _L)]
            m = (vals >= lo_n) & (vals < hi_n)
            q = jnp.where(m, (vals - lo_n) >> _SHIFT, 0)
            cnt, lastm = plsc.scan_count(q, m)
            fl = plsc.load_gather(fills_v, [q], mask=m)
            rank = fl + cnt - 1
            pos = jnp.broadcast_to(g * _L, (_L,)) + lane_iota
            plsc.store_scatter(srt_n, [rank], vals, mask=m)
            plsc.store_scatter(srt_p, [rank], pos, mask=m)
            plsc.addupdate_scatter(fills_v, [q], cnt, mask=lastm & m)
        return carry

    lax.fori_loop(0, _NGRP // 2, place, 0)

    def serve_chunk(t, gg):
        pltpu.make_async_copy(
            memt_hbm.at[:, pl.ds(chunk_base(t), _CHUNK)],
            blk_v.at[t % 2], sem_blk).wait()
        s0 = starts_sm[t]
        s1 = starts_sm[t + 1]
        c0 = chunk_base(t)
        lo_g = s0 >> 4
        hi_g = jnp.where(s1 > s0, (s1 + _L - 1) >> 4, lo_g)

        def serve_group(g, gg):
            r = gg % _RING

            @pl.when(gg >= _RING)
            def _():
                pltpu.make_async_copy(
                    out_hbm.at[pl.ds(0, _L)], ring_v.at[r], sem_out).wait()

            e = jnp.broadcast_to(g * _L, (_L,)) + lane_iota
            gmask = (e >= jnp.broadcast_to(s0, (_L,))) \
                & (e < jnp.broadcast_to(s1, (_L,)))
            vals = srt_n[pl.ds(g * _L, _L)]
            pos = srt_p[pl.ds(g * _L, _L)]
            fv = plsc.all_reduce_ffs(gmask)
            fv16 = jnp.broadcast_to(fv, (_L,)) if fv.ndim == 0 else fv
            c16 = vals - jnp.broadcast_to(c0, (_L,))
            c16 = jnp.where(gmask, c16, jnp.take(
                c16, fv16, mode="wrap"))
            posv = jnp.where(gmask, pos, jnp.take(
                pos, fv16, mode="wrap"))
            buf16 = jnp.broadcast_to(t % 2, (_L,))
            for k in range(_L):
                ck = jnp.broadcast_to(
                    jnp.squeeze(lax.slice(c16, (k,), (k + 1,))), (_L,))
                for j in range(MEMORY_DIM // _L):
                    d16 = lane_iota + j * _L
                    vv = plsc.load_gather(blk_v, [buf16, d16, ck])
                    ring_v[r, k, pl.ds(j * _L, _L)] = vv
            pltpu.async_copy(ring_v.at[r], out_hbm.at[posv], sem_out)
            return gg + 1

        gg = lax.fori_loop(lo_g, hi_g, serve_group, gg)

        @pl.when(t + 2 < n_chunks)
        def _():
            issue(t + 2, t % 2)

        return gg

    gg = lax.fori_loop(0, n_chunks, serve_chunk, jnp.int32(0))

    for k in range(_RING):
        @pl.when(gg > k)
        def _(k=k):
            pltpu.make_async_copy(
                out_hbm.at[pl.ds(0, _L)],
                ring_v.at[(gg - 1 - k) % _RING], sem_out).wait()

    cp_lu.wait()
    pltpu.sync_copy(lu_v, lu_out.at[pl.ds(base, _BPW)])


def kernel(n_id, memory, last_update):
    out, lu_out = _gather_sc(n_id, memory.T, last_update)
    return (out[:, :MEMORY_DIM], lu_out)


# speedup vs baseline: 1.0152x; 1.0152x over previous
import functools

import jax
import jax.numpy as jnp
from jax import lax
from jax.experimental import pallas as pl
from jax.experimental.pallas import tpu as pltpu
from jax.experimental.pallas import tpu_sc as plsc

NUM_NODES = 1000000
MEMORY_DIM = 64
BATCH = 16384

_NC = 2
_NS = 16
_NW = _NC * _NS
_BPW = BATCH // _NW
_L = 16
_CHUNK = 512
_SHIFT = 9
_OUTW = 128
_NUNITS = -(-NUM_NODES // _CHUNK)
_MAXCH = -(-_NUNITS // _NW) + 1
_NGRP = BATCH // _L
_PHYS_END = 1000064
_RING = 4

_mesh = plsc.VectorSubcoreMesh(core_axis_name="c", subcore_axis_name="s")


@functools.partial(
    pl.kernel,
    mesh=_mesh,
    out_type=(
        jax.ShapeDtypeStruct((BATCH, _OUTW), jnp.float32),
        jax.ShapeDtypeStruct((BATCH,), jnp.int32),
    ),
    scratch_types=[
        pltpu.VMEM((BATCH,), jnp.int32),
        pltpu.VMEM((BATCH,), jnp.int32),
        pltpu.VMEM((BATCH,), jnp.int32),
        pltpu.VMEM((2, MEMORY_DIM, _CHUNK), jnp.float32),
        pltpu.VMEM((_RING, _L, _OUTW), jnp.float32),
        pltpu.VMEM((128,), jnp.int32),
        pltpu.VMEM((128,), jnp.int32),
        pltpu.SMEM((128,), jnp.int32),
        pltpu.VMEM((_BPW,), jnp.int32),
        pltpu.VMEM((_BPW,), jnp.int32),
        pltpu.SemaphoreType.DMA,
        pltpu.SemaphoreType.DMA,
        pltpu.SemaphoreType.DMA,
    ],
    compiler_params=pltpu.CompilerParams(needs_layout_passes=False),
)
def _gather_sc(nid_hbm, memt_hbm, lu_hbm, out_hbm, lu_out,
               nid_v, srt_n, srt_p, blk_v, ring_v,
               cnts_v, fills_v, starts_sm, idx_lu, lu_v,
               sem_blk, sem_out, sem_lu):
    wid = lax.axis_index("s") * _NC + lax.axis_index("c")
    base = wid * _BPW
    u_lo = wid * _NUNITS // _NW
    u_hi = (wid + 1) * _NUNITS // _NW
    n_chunks = u_hi - u_lo
    lo_n = u_lo * _CHUNK
    hi_n = u_hi * _CHUNK

    lane_iota = lax.iota(jnp.int32, _L)
    zeros16 = jnp.zeros((_L,), jnp.int32)

    pltpu.sync_copy(nid_hbm.at[pl.ds(base, _BPW)], idx_lu)
    cp_lu = pltpu.async_copy(lu_hbm.at[idx_lu], lu_v, sem_lu)

    pltpu.sync_copy(nid_hbm, nid_v)

    def chunk_base(t):
        c0 = (u_lo + t) * _CHUNK
        return pl.multiple_of(jnp.minimum(c0, _PHYS_END - _CHUNK), 128)

    def issue(t, buf):
        pltpu.async_copy(
            memt_hbm.at[:, pl.ds(chunk_base(t), _CHUNK)], blk_v.at[buf],
            sem_blk)

    issue(0, 0)

    @pl.when(n_chunks > 1)
    def _():
        issue(1, 1)

    for i in range(8):
        cnts_v[pl.ds(i * _L, _L)] = zeros16

    def hist(g2, carry):
        for u in range(2):
            vals = nid_v[pl.ds((g2 * 2 + u) * _L, _L)]
            m = (vals >= lo_n) & (vals < hi_n)
            q = jnp.where(m, (vals - lo_n) >> _SHIFT, 0)
            cnt, lastm = plsc.scan_count(q, m)
            plsc.addupdate_scatter(cnts_v, [q], cnt, mask=lastm & m)
        return carry

    lax.fori_loop(0, _NGRP // 2, hist, 0)

    carry_in = jnp.int32(0)
    totals = []
    for i in range(8):
        ci = cnts_v[pl.ds(i * _L, _L)]
        si = plsc.cumsum(ci)
        ei = si - ci + jnp.broadcast_to(carry_in, (_L,))
        fills_v[pl.ds(i * _L, _L)] = ei
        carry_in = carry_in + jnp.squeeze(lax.slice(si, (_L - 1,), (_L,)))
        totals.append(ei)
    for i in range(8):
        for l in range(_L):
            starts_sm[i * _L + l] = jnp.squeeze(
                lax.slice(totals[i], (l,), (l + 1,)))

    def place(g2, carry):
        for u in range(2):
            g = g2 * 2 + u
            vals = nid_v[pl.ds(g * _L, _L)]
            m = (vals >= lo_n) & (vals < hi_n)
            q = jnp.where(m, (vals - lo_n) >> _SHIFT, 0)
            cnt, lastm = plsc.scan_count(q, m)
            fl = plsc.load_gather(fills_v, [q], mask=m)
            rank = fl + cnt - 1
            pos = jnp.broadcast_to(g * _L, (_L,)) + lane_iota
            plsc.store_scatter(srt_n, [rank], vals, mask=m)
            plsc.store_scatter(srt_p, [rank], pos, mask=m)
            plsc.addupdate_scatter(fills_v, [q], cnt, mask=lastm & m)
        return carry

    lax.fori_loop(0, _NGRP // 2, place, 0)

    def serve_chunk(t, gg):
        pltpu.make_async_copy(
            memt_hbm.at[:, pl.ds(chunk_base(t), _CHUNK)],
            blk_v.at[t % 2], sem_blk).wait()
        s0 = starts_sm[t]
        s1 = starts_sm[t + 1]
        c0 = chunk_base(t)
        lo_g = s0 >> 4
        hi_g = jnp.where(s1 > s0, (s1 + _L - 1) >> 4, lo_g)

        def serve_group(g, gg):
            r = gg % _RING

            @pl.when(gg >= _RING)
            def _():
                pltpu.make_async_copy(
                    out_hbm.at[pl.ds(0, _L)], ring_v.at[r], sem_out).wait()

            e = jnp.broadcast_to(g * _L, (_L,)) + lane_iota
            gmask = (e >= jnp.broadcast_to(s0, (_L,))) \
                & (e < jnp.broadcast_to(s1, (_L,)))
            vals = srt_n[pl.ds(g * _L, _L)]
            pos = srt_p[pl.ds(g * _L, _L)]
            fv = plsc.all_reduce_ffs(gmask)
            fv16 = jnp.broadcast_to(fv, (_L,)) if fv.ndim == 0 else fv
            c16 = vals - jnp.broadcast_to(c0, (_L,))
            c16 = jnp.where(gmask, c16, jnp.take(
                c16, fv16, mode="wrap"))
            posv = jnp.where(gmask, pos, jnp.take(
                pos, fv16, mode="wrap"))
            for d in range(MEMORY_DIM):
                vv = plsc.load_gather(
                    blk_v, [jnp.broadcast_to(jnp.int32(t % 2), (_L,)),
                            jnp.broadcast_to(jnp.int32(d), (_L,)), c16])
                plsc.store_scatter(
                    ring_v, [jnp.broadcast_to(jnp.int32(r), (_L,)),
                             lane_iota,
                             jnp.broadcast_to(jnp.int32(d), (_L,))], vv)
            pltpu.async_copy(ring_v.at[r], out_hbm.at[posv], sem_out)
            return gg + 1

        gg = lax.fori_loop(lo_g, hi_g, serve_group, gg)

        @pl.when(t + 2 < n_chunks)
        def _():
            issue(t + 2, t % 2)

        return gg

    gg = lax.fori_loop(0, n_chunks, serve_chunk, jnp.int32(0))

    for k in range(_RING):
        @pl.when(gg > k)
        def _(k=k):
            pltpu.make_async_copy(
                out_hbm.at[pl.ds(0, _L)],
                ring_v.at[(gg - 1 - k) % _RING], sem_out).wait()

    cp_lu.wait()
    pltpu.sync_copy(lu_v, lu_out.at[pl.ds(base, _BPW)])


def kernel(n_id, memory, last_update):
    out, lu_out = _gather_sc(n_id, memory.T, last_update)
    return (out[:, :MEMORY_DIM], lu_out)
